# baseline (speedup 1.0000x reference)
import jax
import jax.numpy as jnp
from jax.experimental import pallas as pl
from jax.experimental.pallas import tpu as pltpu

_NUM_CHUNKS = 1


def _staged_copy(w_ref, o_ref, scratch, in_sems, out_sems):
    rows = o_ref.shape[0]
    chunk = rows // _NUM_CHUNKS
    in_copies = [
        pltpu.make_async_copy(
            w_ref.at[pl.ds(i * chunk, chunk), :],
            scratch.at[pl.ds(i * chunk, chunk), :],
            in_sems.at[i],
        )
        for i in range(_NUM_CHUNKS)
    ]
    out_copies = [
        pltpu.make_async_copy(
            scratch.at[pl.ds(i * chunk, chunk), :],
            o_ref.at[pl.ds(i * chunk, chunk), :],
            out_sems.at[i],
        )
        for i in range(_NUM_CHUNKS)
    ]
    for c in in_copies:
        c.start()
    for i in range(_NUM_CHUNKS):
        in_copies[i].wait()
        out_copies[i].start()
    for c in out_copies:
        c.wait()


def kernel(x, weight):
    seq_len = x.shape[1]
    hidden = weight.shape[1]
    out = pl.pallas_call(
        _staged_copy,
        in_specs=[pl.BlockSpec(memory_space=pl.ANY)],
        out_specs=pl.BlockSpec(memory_space=pl.ANY),
        out_shape=jax.ShapeDtypeStruct((seq_len, hidden), weight.dtype),
        scratch_shapes=[
            pltpu.VMEM((seq_len, hidden), weight.dtype),
            pltpu.SemaphoreType.DMA((_NUM_CHUNKS,)),
            pltpu.SemaphoreType.DMA((_NUM_CHUNKS,)),
        ],
    )(weight)
    return out[None, :, :]


# speedup vs baseline: 1.0590x; 1.0590x over previous
import jax
import jax.numpy as jnp
from jax.experimental import pallas as pl
from jax.experimental.pallas import tpu as pltpu

_NUM_CHUNKS = 2


def _staged_copy(w_ref, o_ref, scratch, in_sems, out_sems):
    rows = o_ref.shape[1]
    chunk = rows // _NUM_CHUNKS
    in_copies = [
        pltpu.make_async_copy(
            w_ref.at[pl.ds(i * chunk, chunk), :],
            scratch.at[pl.ds(i * chunk, chunk), :],
            in_sems.at[i],
        )
        for i in range(_NUM_CHUNKS)
    ]
    out_copies = [
        pltpu.make_async_copy(
            scratch.at[pl.ds(i * chunk, chunk), :],
            o_ref.at[0, pl.ds(i * chunk, chunk), :],
            out_sems.at[i],
        )
        for i in range(_NUM_CHUNKS)
    ]
    for c in in_copies:
        c.start()
    for i in range(_NUM_CHUNKS):
        in_copies[i].wait()
        out_copies[i].start()
    for c in out_copies:
        c.wait()


def kernel(x, weight):
    seq_len = x.shape[1]
    hidden = weight.shape[1]
    return pl.pallas_call(
        _staged_copy,
        in_specs=[pl.BlockSpec(memory_space=pl.ANY)],
        out_specs=pl.BlockSpec(memory_space=pl.ANY),
        out_shape=jax.ShapeDtypeStruct((1, seq_len, hidden), weight.dtype),
        scratch_shapes=[
            pltpu.VMEM((seq_len, hidden), weight.dtype),
            pltpu.SemaphoreType.DMA((_NUM_CHUNKS,)),
            pltpu.SemaphoreType.DMA((_NUM_CHUNKS,)),
        ],
    )(weight)


# speedup vs baseline: 1.0631x; 1.0039x over previous
import jax
from jax.experimental import pallas as pl
from jax.experimental.pallas import tpu as pltpu

_NUM_CHUNKS = 2


def _staged_copy(w_ref, o_ref, scratch, in_sems, out_sems):
    rows = o_ref.shape[1]
    chunk = rows // _NUM_CHUNKS
    in_copies = [
        pltpu.make_async_copy(
            w_ref.at[pl.ds(i * chunk, chunk), :],
            scratch.at[pl.ds(i * chunk, chunk), :],
            in_sems.at[i],
        )
        for i in range(_NUM_CHUNKS)
    ]
    out_copies = [
        pltpu.make_async_copy(
            scratch.at[pl.ds(i * chunk, chunk), :],
            o_ref.at[0, pl.ds(i * chunk, chunk), :],
            out_sems.at[i],
        )
        for i in range(_NUM_CHUNKS)
    ]
    for c in in_copies:
        c.start()
    for i in range(_NUM_CHUNKS):
        in_copies[i].wait()
        out_copies[i].start()
    for c in out_copies:
        c.wait()


def kernel(x, weight):
    seq_len = x.shape[1]
    hidden = weight.shape[1]
    return pl.pallas_call(
        _staged_copy,
        in_specs=[pl.BlockSpec(memory_space=pl.ANY)],
        out_specs=pl.BlockSpec(memory_space=pl.ANY),
        out_shape=jax.ShapeDtypeStruct((1, seq_len, hidden), weight.dtype),
        scratch_shapes=[
            pltpu.VMEM((seq_len, hidden), weight.dtype),
            pltpu.SemaphoreType.DMA((_NUM_CHUNKS,)),
            pltpu.SemaphoreType.DMA((_NUM_CHUNKS,)),
        ],
    )(weight)
